# Initial kernel scaffold; baseline (speedup 1.0000x reference)
import jax
import jax.numpy as jnp
from jax.experimental import pallas as pl


def kernel(x, position_embedding):
    raise NotImplementedError("write your pallas kernel here")


# speedup vs baseline: 5.0406x; 5.0406x over previous
import jax
import jax.numpy as jnp
from jax.experimental import pallas as pl


def _bcast_body(tab_ref, out_ref):
    out_ref[...] = jnp.broadcast_to(tab_ref[...][None, :, :], out_ref.shape)


def kernel(x, position_embedding):
    batch, seq_len, embed_dim = x.shape
    blk = 512
    grid = (seq_len // blk,)
    return pl.pallas_call(
        _bcast_body,
        grid=grid,
        in_specs=[pl.BlockSpec((blk, embed_dim), lambda i: (i, 0))],
        out_specs=pl.BlockSpec((batch, blk, embed_dim), lambda i: (0, i, 0)),
        out_shape=jax.ShapeDtypeStruct((batch, seq_len, embed_dim),
                                       position_embedding.dtype),
    )(position_embedding[:seq_len])


# speedup vs baseline: 5.1809x; 1.0278x over previous
import jax
import jax.numpy as jnp
from jax.experimental import pallas as pl


def _bcast_body(tab_ref, out_ref):
    out_ref[...] = jnp.broadcast_to(tab_ref[...][None, :, :], out_ref.shape)


def kernel(x, position_embedding):
    batch, seq_len, embed_dim = x.shape
    blk = 1024
    grid = (seq_len // blk,)
    return pl.pallas_call(
        _bcast_body,
        grid=grid,
        in_specs=[pl.BlockSpec((blk, embed_dim), lambda i: (i, 0))],
        out_specs=pl.BlockSpec((batch, blk, embed_dim), lambda i: (0, i, 0)),
        out_shape=jax.ShapeDtypeStruct((batch, seq_len, embed_dim),
                                       position_embedding.dtype),
    )(position_embedding[:seq_len])
